# baseline (speedup 1.0000x reference)
import functools
import math

import jax
import jax.numpy as jnp
from jax import lax
from jax.experimental import pallas as pl
from jax.experimental.pallas import tpu as pltpu
from jax.experimental.pallas import tpu_sc as plsc

D_MODEL = 128
LANES = 16
NUM_CORES = 2
NUM_SUBCORES = 16
NUM_WORKERS = NUM_CORES * NUM_SUBCORES

CHUNK = 128
NBUF = 4
SCALE = math.sqrt(float(D_MODEL))


def _sc_body(rows_per_worker, n_chunks, x_hbm, table_hbm, out_hbm,
             idx_v, rows0, rows1, rows2, rows3, gsem, osem):
    wid = lax.axis_index("s") * NUM_CORES + lax.axis_index("c")
    base = wid * rows_per_worker
    bufs = (rows0, rows1, rows2, rows3)

    pltpu.sync_copy(x_hbm.at[pl.ds(base, rows_per_worker)], idx_v)

    def start_gather(g, buf):
        pltpu.async_copy(
            table_hbm.at[idx_v.at[pl.ds(g * CHUNK, CHUNK)]], buf, gsem)

    def wait_gather(g, buf):
        pltpu.make_async_copy(
            table_hbm.at[idx_v.at[pl.ds(g * CHUNK, CHUNK)]], buf, gsem).wait()

    def start_scatter(g, buf):
        pltpu.async_copy(buf, out_hbm.at[pl.ds(base + g * CHUNK, CHUNK)], osem)

    def wait_scatter(buf):
        pltpu.make_async_copy(buf, out_hbm.at[pl.ds(base, CHUNK)], osem).wait()

    def scale_buf(buf):
        @pl.loop(0, CHUNK, unroll=4)
        def row_loop(i):
            for j in range(D_MODEL // LANES):
                sl = pl.ds(j * LANES, LANES)
                buf[i, sl] = buf[i, sl] * SCALE

    for p in range(NBUF - 1):
        start_gather(p, bufs[p])

    @pl.loop(0, n_chunks, step=NBUF)
    def chunk_loop(g):
        for sub in range(NBUF):
            buf = bufs[sub]
            cur = g + sub
            wait_gather(cur, buf)

            @pl.when(cur + NBUF - 1 < n_chunks)
            def _():
                nxt = bufs[(sub + NBUF - 1) % NBUF]

                @pl.when(cur >= 1)
                def _():
                    wait_scatter(nxt)
                start_gather(cur + NBUF - 1, nxt)

            scale_buf(buf)
            start_scatter(cur, buf)

    for b in bufs:
        wait_scatter(b)


@jax.jit
def _embed(x_flat, table):
    n_rows = x_flat.shape[0]
    rows_per_worker = n_rows // NUM_WORKERS
    n_chunks = rows_per_worker // CHUNK

    mesh = plsc.VectorSubcoreMesh(core_axis_name="c", subcore_axis_name="s")
    body = functools.partial(_sc_body, rows_per_worker, n_chunks)
    return pl.kernel(
        body,
        out_type=jax.ShapeDtypeStruct((n_rows, D_MODEL), jnp.float32),
        mesh=mesh,
        scratch_types=[
            pltpu.VMEM((rows_per_worker,), jnp.int32),
            pltpu.VMEM((CHUNK, D_MODEL), jnp.float32),
            pltpu.VMEM((CHUNK, D_MODEL), jnp.float32),
            pltpu.VMEM((CHUNK, D_MODEL), jnp.float32),
            pltpu.VMEM((CHUNK, D_MODEL), jnp.float32),
            pltpu.SemaphoreType.DMA,
            pltpu.SemaphoreType.DMA,
        ],
    )(x_flat, table)


def kernel(x, table):
    b, h = x.shape
    x_flat = x.reshape(b * h).astype(jnp.int32)
    out = _embed(x_flat, table)
    return out.reshape(b, h, D_MODEL)


# speedup vs baseline: 1.0047x; 1.0047x over previous
import functools
import math

import jax
import jax.numpy as jnp
from jax import lax
from jax.experimental import pallas as pl
from jax.experimental.pallas import tpu as pltpu
from jax.experimental.pallas import tpu_sc as plsc

D_MODEL = 128
LANES = 16
NUM_CORES = 2
NUM_SUBCORES = 16
NUM_WORKERS = NUM_CORES * NUM_SUBCORES

CHUNK = 256
GATHER = 128
NSPLIT = CHUNK // GATHER
NBUF = 3
SCALE = math.sqrt(float(D_MODEL))


def _sc_body(rows_per_worker, n_chunks, x_hbm, table_hbm, out_hbm,
             idx_v, rows0, rows1, rows2, gsem, osem):
    wid = lax.axis_index("s") * NUM_CORES + lax.axis_index("c")
    base = wid * rows_per_worker
    bufs = (rows0, rows1, rows2)

    pltpu.sync_copy(x_hbm.at[pl.ds(base, rows_per_worker)], idx_v)

    def start_gather(g, buf):
        for k in range(NSPLIT):
            pltpu.async_copy(
                table_hbm.at[idx_v.at[pl.ds(g * CHUNK + k * GATHER, GATHER)]],
                buf.at[pl.ds(k * GATHER, GATHER)],
                gsem)

    def wait_gather(g, buf):
        for k in range(NSPLIT):
            pltpu.make_async_copy(
                table_hbm.at[idx_v.at[pl.ds(g * CHUNK + k * GATHER, GATHER)]],
                buf.at[pl.ds(k * GATHER, GATHER)],
                gsem).wait()

    def start_scatter(g, buf):
        pltpu.async_copy(buf, out_hbm.at[pl.ds(base + g * CHUNK, CHUNK)], osem)

    def wait_scatter(buf):
        pltpu.make_async_copy(buf, out_hbm.at[pl.ds(base, CHUNK)], osem).wait()

    def scale_buf(buf):
        @pl.loop(0, CHUNK, unroll=2)
        def row_loop(i):
            for j in range(D_MODEL // LANES):
                sl = pl.ds(j * LANES, LANES)
                buf[i, sl] = buf[i, sl] * SCALE

    def step(cur, buf, nxt):
        wait_gather(cur, buf)

        @pl.when(cur + NBUF - 1 < n_chunks)
        def _():
            @pl.when(cur >= 1)
            def _():
                wait_scatter(nxt)
            start_gather(cur + NBUF - 1, nxt)

        scale_buf(buf)
        start_scatter(cur, buf)

    for p in range(NBUF - 1):
        start_gather(p, bufs[p])

    n_main = (n_chunks // NBUF) * NBUF

    @pl.loop(0, n_main, step=NBUF)
    def chunk_loop(g):
        for sub in range(NBUF):
            step(g + sub, bufs[sub], bufs[(sub + NBUF - 1) % NBUF])

    for cur in range(n_main, n_chunks):
        step(jnp.int32(cur), bufs[cur % NBUF], bufs[(cur + NBUF - 1) % NBUF])

    for b in bufs:
        wait_scatter(b)


@jax.jit
def _embed(x_flat, table):
    n_rows = x_flat.shape[0]
    rows_per_worker = n_rows // NUM_WORKERS
    n_chunks = rows_per_worker // CHUNK

    mesh = plsc.VectorSubcoreMesh(core_axis_name="c", subcore_axis_name="s")
    body = functools.partial(_sc_body, rows_per_worker, n_chunks)
    return pl.kernel(
        body,
        out_type=jax.ShapeDtypeStruct((n_rows, D_MODEL), jnp.float32),
        mesh=mesh,
        scratch_types=[
            pltpu.VMEM((rows_per_worker,), jnp.int32),
            pltpu.VMEM((CHUNK, D_MODEL), jnp.float32),
            pltpu.VMEM((CHUNK, D_MODEL), jnp.float32),
            pltpu.VMEM((CHUNK, D_MODEL), jnp.float32),
            pltpu.SemaphoreType.DMA,
            pltpu.SemaphoreType.DMA,
        ],
    )(x_flat, table)


def kernel(x, table):
    b, h = x.shape
    x_flat = x.reshape(b * h).astype(jnp.int32)
    out = _embed(x_flat, table)
    return out.reshape(b, h, D_MODEL)


# speedup vs baseline: 1.0180x; 1.0133x over previous
import functools
import math

import jax
import jax.numpy as jnp
from jax import lax
from jax.experimental import pallas as pl
from jax.experimental.pallas import tpu as pltpu
from jax.experimental.pallas import tpu_sc as plsc

D_MODEL = 128
LANES = 16
NUM_CORES = 2
NUM_SUBCORES = 16
NUM_WORKERS = NUM_CORES * NUM_SUBCORES

CHUNK = 256
GATHER = 128
NSPLIT = CHUNK // GATHER
SCALE = math.sqrt(float(D_MODEL))


def _sc_body(rows_per_worker, n_chunks, x_hbm, table_hbm, out_hbm,
             idx_v, rows0, rows1, gsem, osem):
    wid = lax.axis_index("s") * NUM_CORES + lax.axis_index("c")
    base = wid * rows_per_worker
    bufs = (rows0, rows1)

    pltpu.sync_copy(x_hbm.at[pl.ds(base, rows_per_worker)], idx_v)

    def start_gather(g, buf):
        for k in range(NSPLIT):
            pltpu.async_copy(
                table_hbm.at[idx_v.at[pl.ds(g * CHUNK + k * GATHER, GATHER)]],
                buf.at[pl.ds(k * GATHER, GATHER)],
                gsem)

    def wait_gather(g, buf):
        for k in range(NSPLIT):
            pltpu.make_async_copy(
                table_hbm.at[idx_v.at[pl.ds(g * CHUNK + k * GATHER, GATHER)]],
                buf.at[pl.ds(k * GATHER, GATHER)],
                gsem).wait()

    def start_scatter(g, buf):
        pltpu.async_copy(buf, out_hbm.at[pl.ds(base + g * CHUNK, CHUNK)], osem)

    def wait_scatter(buf):
        pltpu.make_async_copy(buf, out_hbm.at[pl.ds(base, CHUNK)], osem).wait()

    def scale_buf(buf):
        @pl.loop(0, CHUNK)
        def row_loop(i):
            for j in range(D_MODEL // LANES):
                sl = pl.ds(j * LANES, LANES)
                buf[i, sl] = buf[i, sl] * SCALE

    start_gather(0, rows0)

    @pl.loop(0, n_chunks, step=2)
    def chunk_loop(g):
        for sub in range(2):
            buf = bufs[sub]
            other = bufs[1 - sub]
            cur = g + sub
            wait_gather(cur, buf)

            @pl.when(cur + 1 < n_chunks)
            def _():
                @pl.when(cur >= 1)
                def _():
                    wait_scatter(other)
                start_gather(cur + 1, other)

            scale_buf(buf)
            start_scatter(cur, buf)

    wait_scatter(rows0)
    wait_scatter(rows1)


@jax.jit
def _embed(x_flat, table):
    n_rows = x_flat.shape[0]
    rows_per_worker = n_rows // NUM_WORKERS
    n_chunks = rows_per_worker // CHUNK

    mesh = plsc.VectorSubcoreMesh(core_axis_name="c", subcore_axis_name="s")
    body = functools.partial(_sc_body, rows_per_worker, n_chunks)
    return pl.kernel(
        body,
        out_type=jax.ShapeDtypeStruct((n_rows, D_MODEL), jnp.float32),
        mesh=mesh,
        scratch_types=[
            pltpu.VMEM((rows_per_worker,), jnp.int32),
            pltpu.VMEM((CHUNK, D_MODEL), jnp.float32),
            pltpu.VMEM((CHUNK, D_MODEL), jnp.float32),
            pltpu.SemaphoreType.DMA,
            pltpu.SemaphoreType.DMA,
        ],
    )(x_flat, table)


def kernel(x, table):
    b, h = x.shape
    x_flat = x.reshape(b * h).astype(jnp.int32)
    out = _embed(x_flat, table)
    return out.reshape(b, h, D_MODEL)
